# baseline (speedup 1.0000x reference)
import functools

import jax
import jax.numpy as jnp
from jax import lax
from jax.experimental import pallas as pl
from jax.experimental.pallas import tpu as pltpu
from jax.experimental.pallas import tpu_sc as plsc


_L = 16


def _make_pool(B, H, V, D, nc, ns):
    nw = nc * ns
    bpw = B // nw
    bpc = 2
    CH = bpc * H
    nch = bpw // bpc
    nd = D // _L
    inv = 1.0 / H

    mesh = plsc.VectorSubcoreMesh(core_axis_name="c", subcore_axis_name="s")

    @functools.partial(
        pl.kernel,
        mesh=mesh,
        compiler_params=pltpu.CompilerParams(use_tc_tiling_on_sc=False),
        out_type=jax.ShapeDtypeStruct((B, D), jnp.float32),
        scratch_types=[
            pltpu.VMEM((nch, CH), jnp.int32),
            pltpu.VMEM((bpw * H, D), jnp.float32),
            pltpu.VMEM((bpw, D), jnp.float32),
            pltpu.SemaphoreType.DMA,
        ],
    )
    def pool(ids_hbm, table_hbm, out_hbm, idx_v, rows_v, pool_v, sem):
        wid = lax.axis_index("s") * nc + lax.axis_index("c")
        pltpu.sync_copy(ids_hbm.at[pl.ds(wid * nch, nch)], idx_v)
        cps = [
            pltpu.async_copy(
                table_hbm.at[idx_v.at[c]],
                rows_v.at[pl.ds(c * CH, CH)],
                sem,
            )
            for c in range(nch)
        ]
        for cp in cps:
            cp.wait()
        zero = jnp.zeros((_L,), jnp.float32)
        for i in range(bpw):
            r0 = i * H

            def red(j, accs, r0=r0):
                return tuple(
                    accs[t] + rows_v[r0 + j, pl.ds(t * _L, _L)]
                    for t in range(nd)
                )

            accs = lax.fori_loop(0, H, red, (zero,) * nd)
            for t in range(nd):
                pool_v[i, pl.ds(t * _L, _L)] = accs[t] * inv
        pltpu.sync_copy(pool_v, out_hbm.at[pl.ds(wid * bpw, bpw)])

    return pool


_BS = 32


def _proj_body(avg_ref, w_ref, b_ref, out_ref):
    out_ref[...] = (
        lax.dot_general(
            avg_ref[...],
            w_ref[...],
            dimension_numbers=(((1,), (1,)), ((), ())),
            preferred_element_type=jnp.float32,
        )
        + b_ref[...]
    )


def _make_proj(B, D, V):
    return pl.pallas_call(
        _proj_body,
        grid=(B // _BS,),
        in_specs=[
            pl.BlockSpec((_BS, D), lambda i: (i, 0)),
            pl.BlockSpec((V, D), lambda i: (0, 0)),
            pl.BlockSpec((1, V), lambda i: (0, 0)),
        ],
        out_specs=pl.BlockSpec((_BS, V), lambda i: (i, 0)),
        out_shape=jax.ShapeDtypeStruct((B, V), jnp.float32),
        compiler_params=pltpu.CompilerParams(
            dimension_semantics=("arbitrary",),
        ),
    )


def kernel(context_ids, emb_table, W, b):
    B, H = context_ids.shape
    V, D = emb_table.shape
    info = plsc.get_sparse_core_info()
    nc, ns = info.num_cores, info.num_subcores
    bpc = 2
    ids2 = context_ids.astype(jnp.int32).reshape(B // bpc, bpc * H)
    pooled = _make_pool(B, H, V, D, nc, ns)(ids2, emb_table)
    return _make_proj(B, D, V)(
        pooled.astype(jnp.bfloat16),
        W.astype(jnp.bfloat16),
        b.reshape(1, V),
    )


# speedup vs baseline: 1.3190x; 1.3190x over previous
import functools

import jax
import jax.numpy as jnp
from jax import lax
from jax.experimental import pallas as pl
from jax.experimental.pallas import tpu as pltpu
from jax.experimental.pallas import tpu_sc as plsc


_L = 16


def _make_pool(B, H, V, D, nc, ns):
    nw = nc * ns
    bpw = B // nw
    bpc = 2
    CH = bpc * H
    nch = bpw // bpc
    nd = D // _L
    inv = 1.0 / H

    mesh = plsc.VectorSubcoreMesh(core_axis_name="c", subcore_axis_name="s")

    @functools.partial(
        pl.kernel,
        mesh=mesh,
        compiler_params=pltpu.CompilerParams(use_tc_tiling_on_sc=False),
        out_type=jax.ShapeDtypeStruct((B, D), jnp.float32),
        scratch_types=[
            pltpu.VMEM((nch, CH), jnp.int32),
            pltpu.VMEM((bpw * H, D), jnp.float32),
            pltpu.VMEM((bpw, D), jnp.float32),
            pltpu.SemaphoreType.DMA,
        ],
    )
    def pool(ids_hbm, table_hbm, out_hbm, idx_v, rows_v, pool_v, sem):
        wid = lax.axis_index("s") * nc + lax.axis_index("c")
        pltpu.sync_copy(ids_hbm.at[pl.ds(wid * nch, nch)], idx_v)
        cps = [
            pltpu.async_copy(
                table_hbm.at[idx_v.at[c]],
                rows_v.at[pl.ds(c * CH, CH)],
                sem,
            )
            for c in range(nch)
        ]
        for cp in cps:
            cp.wait()
        zero = jnp.zeros((_L,), jnp.float32)
        for i in range(bpw):
            r0 = i * H

            def red(j, accs, r0=r0):
                return tuple(
                    accs[t] + rows_v[r0 + j, pl.ds(t * _L, _L)]
                    for t in range(nd)
                )

            accs = lax.fori_loop(0, H, red, (zero,) * nd)
            for t in range(nd):
                pool_v[i, pl.ds(t * _L, _L)] = accs[t] * inv
        pltpu.sync_copy(pool_v, out_hbm.at[pl.ds(wid * bpw, bpw)])

    return pool


_BS = 256
_VT = 12544


def _make_proj(B, D, V):
    nb = B // _BS
    nv = (V + _VT - 1) // _VT
    Vp = nv * _VT

    def body(avg_ref, wt_hbm, b_hbm, o_ref, wt_v, b_v):
        i = pl.program_id(0)
        j = pl.program_id(1)

        @pl.when(jnp.logical_and(i == 0, j == 0))
        def _():
            pltpu.sync_copy(wt_hbm, wt_v)
            pltpu.sync_copy(b_hbm, b_v)

        c = j * _VT
        o_ref[...] = (
            lax.dot_general(
                avg_ref[...],
                wt_v[:, pl.ds(c, _VT)],
                dimension_numbers=(((1,), (0,)), ((), ())),
                preferred_element_type=jnp.float32,
            )
            + b_v[:, pl.ds(c, _VT)]
        )

    return pl.pallas_call(
        body,
        grid=(nb, nv),
        in_specs=[
            pl.BlockSpec((_BS, D), lambda i, j: (i, 0)),
            pl.BlockSpec(memory_space=pl.ANY),
            pl.BlockSpec(memory_space=pl.ANY),
        ],
        out_specs=pl.BlockSpec((_BS, _VT), lambda i, j: (i, j)),
        out_shape=jax.ShapeDtypeStruct((B, V), jnp.float32),
        scratch_shapes=[
            pltpu.VMEM((D, Vp), jnp.bfloat16),
            pltpu.VMEM((1, Vp), jnp.float32),
        ],
        compiler_params=pltpu.CompilerParams(
            dimension_semantics=("arbitrary", "arbitrary"),
        ),
    )


def kernel(context_ids, emb_table, W, b):
    B, H = context_ids.shape
    V, D = emb_table.shape
    info = plsc.get_sparse_core_info()
    nc, ns = info.num_cores, info.num_subcores
    bpc = 2
    ids2 = context_ids.astype(jnp.int32).reshape(B // bpc, bpc * H)
    pooled = _make_pool(B, H, V, D, nc, ns)(ids2, emb_table)
    nv = (V + _VT - 1) // _VT
    Vp = nv * _VT
    wt = jnp.pad(W.T.astype(jnp.bfloat16), ((0, 0), (0, Vp - V)))
    bp = jnp.pad(b, (0, Vp - V)).reshape(1, Vp)
    return _make_proj(B, D, V)(
        pooled.astype(jnp.bfloat16),
        wt,
        bp,
    )


# speedup vs baseline: 1.3202x; 1.0009x over previous
import functools

import jax
import jax.numpy as jnp
from jax import lax
from jax.experimental import pallas as pl
from jax.experimental.pallas import tpu as pltpu
from jax.experimental.pallas import tpu_sc as plsc


_L = 16


def _make_pool(B, H, V, D, nc, ns):
    nw = nc * ns
    bpw = B // nw
    bpc = 2
    CH = bpc * H
    nch = bpw // bpc
    nd = D // _L
    inv = 1.0 / H

    mesh = plsc.VectorSubcoreMesh(core_axis_name="c", subcore_axis_name="s")

    @functools.partial(
        pl.kernel,
        mesh=mesh,
        compiler_params=pltpu.CompilerParams(use_tc_tiling_on_sc=False),
        out_type=jax.ShapeDtypeStruct((B, D), jnp.float32),
        scratch_types=[
            pltpu.VMEM((nch, CH), jnp.int32),
            pltpu.VMEM((bpw * H, D), jnp.float32),
            pltpu.VMEM((bpw, D), jnp.float32),
            pltpu.SemaphoreType.DMA,
        ],
    )
    def pool(ids_hbm, table_hbm, out_hbm, idx_v, rows_v, pool_v, sem):
        wid = lax.axis_index("s") * nc + lax.axis_index("c")
        pltpu.sync_copy(ids_hbm.at[pl.ds(wid * nch, nch)], idx_v)
        cps = [
            pltpu.async_copy(
                table_hbm.at[idx_v.at[c]],
                rows_v.at[pl.ds(c * CH, CH)],
                sem,
            )
            for c in range(nch)
        ]
        for cp in cps:
            cp.wait()
        zero = jnp.zeros((_L,), jnp.float32)
        for i in range(bpw):
            r0 = i * H

            def red(j, accs, r0=r0):
                return tuple(
                    accs[t] + rows_v[r0 + j, pl.ds(t * _L, _L)]
                    for t in range(nd)
                )

            accs = lax.fori_loop(0, H, red, (zero,) * nd)
            for t in range(nd):
                pool_v[i, pl.ds(t * _L, _L)] = accs[t] * inv
        pltpu.sync_copy(pool_v, out_hbm.at[pl.ds(wid * bpw, bpw)])

    return pool


_BS = 256
_VT = 12544


def _make_proj(B, D, V):
    nb = B // _BS
    nv = (V + _VT - 1) // _VT

    def body(avg_ref, wt_ref, b_ref, o_ref):
        o_ref[...] = (
            lax.dot_general(
                avg_ref[...],
                wt_ref[...],
                dimension_numbers=(((1,), (0,)), ((), ())),
                preferred_element_type=jnp.float32,
            )
            + b_ref[...]
        )

    return pl.pallas_call(
        body,
        grid=(nv, nb),
        in_specs=[
            pl.BlockSpec((_BS, D), lambda j, i: (i, 0)),
            pl.BlockSpec((D, _VT), lambda j, i: (0, j)),
            pl.BlockSpec((1, _VT), lambda j, i: (0, j)),
        ],
        out_specs=pl.BlockSpec((_BS, _VT), lambda j, i: (i, j)),
        out_shape=jax.ShapeDtypeStruct((B, V), jnp.float32),
        compiler_params=pltpu.CompilerParams(
            dimension_semantics=("parallel", "parallel"),
        ),
    )


def kernel(context_ids, emb_table, W, b):
    B, H = context_ids.shape
    V, D = emb_table.shape
    info = plsc.get_sparse_core_info()
    nc, ns = info.num_cores, info.num_subcores
    bpc = 2
    ids2 = context_ids.astype(jnp.int32).reshape(B // bpc, bpc * H)
    pooled = _make_pool(B, H, V, D, nc, ns)(ids2, emb_table)
    nv = (V + _VT - 1) // _VT
    Vp = nv * _VT
    wt = jnp.pad(W.T.astype(jnp.bfloat16), ((0, 0), (0, Vp - V)))
    bp = jnp.pad(b, (0, Vp - V)).reshape(1, Vp)
    return _make_proj(B, D, V)(
        pooled.astype(jnp.bfloat16),
        wt,
        bp,
    )
